# baseline (speedup 1.0000x reference)
import functools
import math

import jax
import jax.numpy as jnp
from jax import lax
from jax.experimental import pallas as pl
from jax.experimental.pallas import tpu as pltpu
from jax.experimental.pallas import tpu_sc as plsc

_THRESH = 0.6
_MIN_KEPT = 100000

_NEG_LOG_THRESH = -math.log(_THRESH)

_N = 4 * 512 * 512
_RANK = _N - _MIN_KEPT + 1
_NT = 16
_PER = _N // _NT
_NV = _PER // 16
_NBIN = 2048


def _nll_kernel(pred_ref, tgt_ref, nll_ref):
    x = pred_ref[...]
    m = jnp.max(x, axis=1, keepdims=True)
    sh = x - m
    s = jnp.sum(jnp.exp(sh), axis=1)
    t = tgt_ref[...]
    cls = jax.lax.broadcasted_iota(jnp.int32, x.shape, 1)
    sh_t = jnp.sum(jnp.where(cls == t[:, None], sh, 0.0), axis=1)
    nll_ref[...] = jnp.log(s) - sh_t


def _keys(v):
    b = lax.bitcast_convert_type(v, jnp.int32)
    return b ^ ((b >> 31) & jnp.int32(0x7FFFFFFF))


def _sc_select_body(nll_hbm, out_hbm, data_v, hist_v, merge_v, out_v,
                    sh_hist):
    tid = lax.axis_index("s")
    lio = lax.iota(jnp.int32, 16)
    zeros_i = jnp.zeros((16,), jnp.int32)
    ones_i = jnp.ones((16,), jnp.int32)

    pltpu.sync_copy(nll_hbm.at[pl.ds(tid * _PER, _PER)], data_v)

    def zero_hist(i, _):
        hist_v[pl.ds(i * 16, 16)] = zeros_i
        return 0

    def hist_pass(rnd, p1, p2):
        lax.fori_loop(0, _NBIN // 16, zero_hist, 0)

        def body(i, _):
            key = _keys(data_v[pl.ds(i * 16, 16)])
            if rnd == 0:
                bin_ = (key >> 21) + 1024
                plsc.addupdate_scatter(hist_v, [bin_], ones_i)
            elif rnd == 1:
                bin_ = (key >> 10) & jnp.int32(0x7FF)
                plsc.addupdate_scatter(hist_v, [bin_], ones_i,
                                       mask=(key >> 21) == p1)
            else:
                bin_ = key & jnp.int32(0x3FF)
                plsc.addupdate_scatter(hist_v, [bin_], ones_i,
                                       mask=(key >> 10) == p2)
            return 0

        lax.fori_loop(0, _NV, body, 0)

        pltpu.sync_copy(hist_v, sh_hist.at[tid])
        plsc.subcore_barrier()
        pltpu.sync_copy(sh_hist, merge_v)
        plsc.subcore_barrier()
        return None

    def find_bin(rank):
        def scan(j, carry):
            cum, b_acc, below_acc = carry
            h = zeros_i
            for t in range(_NT):
                h = h + merge_v[t, pl.ds(j * 16, 16)]
            c = plsc.cumsum(h)
            tot = jnp.sum(h)
            cumv = cum + c
            excl = cumv - h
            hit = jnp.logical_and(cumv >= rank, excl < rank)
            b_acc = b_acc + jnp.sum(jnp.where(hit, j * 16 + lio, 0))
            below_acc = below_acc + jnp.sum(jnp.where(hit, excl, 0))
            return (cum + tot, b_acc, below_acc)

        _, b, below = lax.fori_loop(
            0, _NBIN // 16, scan,
            (jnp.int32(0), jnp.int32(0), jnp.int32(0)))
        return b, below

    hist_pass(0, None, None)
    b1, below1 = find_bin(jnp.int32(_RANK))
    p1 = b1 - 1024
    rank2 = jnp.int32(_RANK) - below1

    hist_pass(1, p1, None)
    b2, below2 = find_bin(rank2)
    p2 = (p1 << 11) | b2
    rank3 = rank2 - below2

    hist_pass(2, None, p2)
    b3, _ = find_bin(rank3)
    key_fin = (p2 << 10) | b3

    kf = key_fin + zeros_i
    tb = jnp.where(kf >= 0, kf, kf ^ jnp.int32(0x7FFFFFFF))
    tnll = plsc.bitcast(tb, jnp.float32)
    thr = jnp.minimum(tnll, jnp.float32(_NEG_LOG_THRESH))

    def loss_body(i, carry):
        s, cnt = carry
        v = data_v[pl.ds(i * 16, 16)]
        kept = v >= thr
        s = s + jnp.where(kept, v, 0.0)
        cnt = cnt + jnp.where(kept, 1, 0)
        return (s, cnt)

    s_v, c_v = lax.fori_loop(
        0, _NV, loss_body,
        (jnp.zeros((16,), jnp.float32), zeros_i))

    hist_v[pl.ds(0, 16)] = lax.bitcast_convert_type(s_v, jnp.int32)
    hist_v[pl.ds(16, 16)] = c_v
    pltpu.sync_copy(hist_v, sh_hist.at[tid])
    plsc.subcore_barrier()

    @pl.when(tid == 0)
    def _():
        pltpu.sync_copy(sh_hist, merge_v)
        sf = jnp.zeros((16,), jnp.float32)
        ci = zeros_i
        for t in range(_NT):
            sf = sf + lax.bitcast_convert_type(
                merge_v[t, pl.ds(0, 16)], jnp.float32)
            ci = ci + merge_v[t, pl.ds(16, 16)]
        num_v = jnp.sum(sf) + jnp.zeros((16,), jnp.float32)
        cnt_v2 = jnp.sum(ci) + zeros_i
        den_v = jnp.maximum(cnt_v2.astype(jnp.float32), 1.0)
        out_v[...] = num_v / den_v
        pltpu.sync_copy(out_v, out_hbm)


def kernel(pred, target):
    b, c, h, w = pred.shape
    hc = 16

    nll = pl.pallas_call(
        _nll_kernel,
        grid=(h // hc,),
        in_specs=[
            pl.BlockSpec((b, c, hc, w), lambda i: (0, 0, i, 0)),
            pl.BlockSpec((b, hc, w), lambda i: (0, i, 0)),
        ],
        out_specs=pl.BlockSpec((b, hc, w), lambda i: (0, i, 0)),
        out_shape=jax.ShapeDtypeStruct((b, h, w), jnp.float32),
    )(pred, target)

    mesh = plsc.VectorSubcoreMesh(
        core_axis_name="c", subcore_axis_name="s", num_cores=1)
    sc_select = pl.kernel(
        _sc_select_body,
        mesh=mesh,
        compiler_params=pltpu.CompilerParams(needs_layout_passes=False),
        out_type=jax.ShapeDtypeStruct((16,), jnp.float32),
        scratch_types=[
            pltpu.VMEM((_PER,), jnp.float32),
            pltpu.VMEM((_NBIN,), jnp.int32),
            pltpu.VMEM((_NT, _NBIN), jnp.int32),
            pltpu.VMEM((16,), jnp.float32),
            pltpu.VMEM_SHARED((_NT, _NBIN), jnp.int32),
        ],
    )
    loss16 = sc_select(nll.reshape(-1))
    return loss16[0]


# speedup vs baseline: 1.0776x; 1.0776x over previous
import functools
import math

import jax
import jax.numpy as jnp
from jax import lax
from jax.experimental import pallas as pl
from jax.experimental.pallas import tpu as pltpu
from jax.experimental.pallas import tpu_sc as plsc

_THRESH = 0.6
_MIN_KEPT = 100000

_NEG_LOG_THRESH = -math.log(_THRESH)

_N = 4 * 512 * 512
_RANK = _N - _MIN_KEPT + 1
_NT = 16
_PER = _N // _NT
_NV = _PER // 16
_NBIN = 2048
_UNROLL = 4


def _nll_kernel(pred_ref, tgt_ref, nll_ref):
    x = pred_ref[...]
    m = jnp.max(x, axis=1, keepdims=True)
    sh = x - m
    s = jnp.sum(jnp.exp(sh), axis=1)
    t = tgt_ref[...]
    cls = jax.lax.broadcasted_iota(jnp.int32, x.shape, 1)
    sh_t = jnp.sum(jnp.where(cls == t[:, None], sh, 0.0), axis=1)
    nll_ref[...] = jnp.log(s) - sh_t


def _keys(v):
    b = lax.bitcast_convert_type(v, jnp.int32)
    return b ^ ((b >> 31) & jnp.int32(0x7FFFFFFF))


def _sc_select_body(nll_hbm, out_hbm, data_v, hist_v, merge_v, out_v,
                    sh_hist):
    tid = lax.axis_index("s")
    lio = lax.iota(jnp.int32, 16)
    zeros_i = jnp.zeros((16,), jnp.int32)
    ones_i = jnp.ones((16,), jnp.int32)

    pltpu.sync_copy(nll_hbm.at[pl.ds(tid * _PER, _PER)], data_v)

    def zero_hist(i, _):
        hist_v[pl.ds(i * 16, 16)] = zeros_i
        return 0

    def hist_pass(rnd, p1, p2):
        lax.fori_loop(0, _NBIN // 16, zero_hist, 0)

        def body(i, _):
            for u in range(_UNROLL):
                key = _keys(data_v[pl.ds(i * (16 * _UNROLL) + u * 16, 16)])
                if rnd == 0:
                    bin_ = (key >> 21) + 1024
                    plsc.addupdate_scatter(hist_v, [bin_], ones_i)
                elif rnd == 1:
                    bin_ = (key >> 10) & jnp.int32(0x7FF)
                    plsc.addupdate_scatter(hist_v, [bin_], ones_i,
                                           mask=(key >> 21) == p1)
                else:
                    bin_ = key & jnp.int32(0x3FF)
                    plsc.addupdate_scatter(hist_v, [bin_], ones_i,
                                           mask=(key >> 10) == p2)
            return 0

        lax.fori_loop(0, _NV // _UNROLL, body, 0)

        pltpu.sync_copy(hist_v, sh_hist.at[tid])
        plsc.subcore_barrier()
        pltpu.sync_copy(sh_hist, merge_v)
        plsc.subcore_barrier()
        return None

    def find_bin(rank):
        def scan(j, carry):
            cum, b_acc, below_acc = carry
            h = zeros_i
            for t in range(_NT):
                h = h + merge_v[t, pl.ds(j * 16, 16)]
            c = plsc.cumsum(h)
            tot = jnp.sum(h)
            cumv = cum + c
            excl = cumv - h
            hit = jnp.logical_and(cumv >= rank, excl < rank)
            b_acc = b_acc + jnp.sum(jnp.where(hit, j * 16 + lio, 0))
            below_acc = below_acc + jnp.sum(jnp.where(hit, excl, 0))
            return (cum + tot, b_acc, below_acc)

        _, b, below = lax.fori_loop(
            0, _NBIN // 16, scan,
            (jnp.int32(0), jnp.int32(0), jnp.int32(0)))
        return b, below

    hist_pass(0, None, None)
    b1, below1 = find_bin(jnp.int32(_RANK))
    p1 = b1 - 1024
    rank2 = jnp.int32(_RANK) - below1

    hist_pass(1, p1, None)
    b2, below2 = find_bin(rank2)
    p2 = (p1 << 11) | b2
    rank3 = rank2 - below2

    hist_pass(2, None, p2)
    b3, _ = find_bin(rank3)
    key_fin = (p2 << 10) | b3

    kf = key_fin + zeros_i
    tb = jnp.where(kf >= 0, kf, kf ^ jnp.int32(0x7FFFFFFF))
    tnll = plsc.bitcast(tb, jnp.float32)
    thr = jnp.minimum(tnll, jnp.float32(_NEG_LOG_THRESH))

    def loss_body(i, carry):
        s, cnt = carry
        for u in range(_UNROLL):
            v = data_v[pl.ds(i * (16 * _UNROLL) + u * 16, 16)]
            kept = v >= thr
            s = s + jnp.where(kept, v, 0.0)
            cnt = cnt + jnp.where(kept, 1, 0)
        return (s, cnt)

    s_v, c_v = lax.fori_loop(
        0, _NV // _UNROLL, loss_body,
        (jnp.zeros((16,), jnp.float32), zeros_i))

    hist_v[pl.ds(0, 16)] = lax.bitcast_convert_type(s_v, jnp.int32)
    hist_v[pl.ds(16, 16)] = c_v
    pltpu.sync_copy(hist_v, sh_hist.at[tid])
    plsc.subcore_barrier()

    @pl.when(tid == 0)
    def _():
        pltpu.sync_copy(sh_hist, merge_v)
        sf = jnp.zeros((16,), jnp.float32)
        ci = zeros_i
        for t in range(_NT):
            sf = sf + lax.bitcast_convert_type(
                merge_v[t, pl.ds(0, 16)], jnp.float32)
            ci = ci + merge_v[t, pl.ds(16, 16)]
        num_v = jnp.sum(sf) + jnp.zeros((16,), jnp.float32)
        cnt_v2 = jnp.sum(ci) + zeros_i
        den_v = jnp.maximum(cnt_v2.astype(jnp.float32), 1.0)
        out_v[...] = num_v / den_v
        pltpu.sync_copy(out_v, out_hbm)


def kernel(pred, target):
    b, c, h, w = pred.shape
    hc = 16

    nll = pl.pallas_call(
        _nll_kernel,
        grid=(h // hc,),
        in_specs=[
            pl.BlockSpec((b, c, hc, w), lambda i: (0, 0, i, 0)),
            pl.BlockSpec((b, hc, w), lambda i: (0, i, 0)),
        ],
        out_specs=pl.BlockSpec((b, hc, w), lambda i: (0, i, 0)),
        out_shape=jax.ShapeDtypeStruct((b, h, w), jnp.float32),
    )(pred, target)

    mesh = plsc.VectorSubcoreMesh(
        core_axis_name="c", subcore_axis_name="s", num_cores=1)
    sc_select = pl.kernel(
        _sc_select_body,
        mesh=mesh,
        compiler_params=pltpu.CompilerParams(needs_layout_passes=False),
        out_type=jax.ShapeDtypeStruct((16,), jnp.float32),
        scratch_types=[
            pltpu.VMEM((_PER,), jnp.float32),
            pltpu.VMEM((_NBIN,), jnp.int32),
            pltpu.VMEM((_NT, _NBIN), jnp.int32),
            pltpu.VMEM((16,), jnp.float32),
            pltpu.VMEM_SHARED((_NT, _NBIN), jnp.int32),
        ],
    )
    loss16 = sc_select(nll.reshape(-1))
    return loss16[0]


# speedup vs baseline: 1.1054x; 1.0258x over previous
import functools
import math

import jax
import jax.numpy as jnp
from jax import lax
from jax.experimental import pallas as pl
from jax.experimental.pallas import tpu as pltpu
from jax.experimental.pallas import tpu_sc as plsc

_THRESH = 0.6
_MIN_KEPT = 100000

_NEG_LOG_THRESH = -math.log(_THRESH)

_N = 4 * 512 * 512
_RANK = _N - _MIN_KEPT + 1
_NT = 16
_PER = _N // _NT
_NV = _PER // 16
_NBIN = 2048
_UNROLL = 4


def _nll_kernel(pred_ref, tgt_ref, nll_ref):
    x = pred_ref[...]
    m = jnp.max(x, axis=1, keepdims=True)
    sh = x - m
    s = jnp.sum(jnp.exp(sh), axis=1)
    t = tgt_ref[...]
    cls = jax.lax.broadcasted_iota(jnp.int32, x.shape, 1)
    sh_t = jnp.sum(jnp.where(cls == t[:, None], sh, 0.0), axis=1)
    nll_ref[...] = jnp.log(s) - sh_t


def _keys(v):
    b = lax.bitcast_convert_type(v, jnp.int32)
    return b ^ ((b >> 31) & jnp.int32(0x7FFFFFFF))


def _sc_select_body(nll_hbm, out_hbm, data_v, hist_v, out_v,
                    strip_v, merged_v, stage_v, tot_v, res_v,
                    sh_hist, sh_aux):
    tid = lax.axis_index("s")
    lio = lax.iota(jnp.int32, 16)
    zeros_i = jnp.zeros((16,), jnp.int32)
    ones_i = jnp.ones((16,), jnp.int32)
    zeros_f = jnp.zeros((16,), jnp.float32)
    _SUMS, _CNTS, _TOTS, _RES = 0, 256, 512, 768

    pltpu.sync_copy(nll_hbm.at[pl.ds(tid * _PER, _PER)], data_v)

    def zero_hist(i, _):
        hist_v[pl.ds(i * 16, 16)] = zeros_i
        return 0

    def hist_pass(rnd, p1, p2, rank):
        lax.fori_loop(0, _NBIN // 16, zero_hist, 0)

        def body(i, _):
            for u in range(_UNROLL):
                key = _keys(data_v[pl.ds(i * (16 * _UNROLL) + u * 16, 16)])
                if rnd == 0:
                    bin_ = (key >> 21) + 1024
                    plsc.addupdate_scatter(hist_v, [bin_], ones_i)
                elif rnd == 1:
                    bin_ = (key >> 10) & jnp.int32(0x7FF)
                    plsc.addupdate_scatter(hist_v, [bin_], ones_i,
                                           mask=(key >> 21) == p1)
                else:
                    bin_ = key & jnp.int32(0x3FF)
                    plsc.addupdate_scatter(hist_v, [bin_], ones_i,
                                           mask=(key >> 10) == p2)
            return 0

        lax.fori_loop(0, _NV // _UNROLL, body, 0)

        pltpu.sync_copy(hist_v, sh_hist.at[tid])
        plsc.subcore_barrier()
        pltpu.sync_copy(sh_hist.at[:, pl.ds(tid * 128, 128)], strip_v)
        stot = jnp.int32(0)
        for ch in range(8):
            h = zeros_i
            for r in range(_NT):
                h = h + strip_v[r, pl.ds(ch * 16, 16)]
            merged_v[pl.ds(ch * 16, 16)] = h
            stot = stot + jnp.sum(h)
        stage_v[...] = stot + zeros_i
        pltpu.sync_copy(stage_v, sh_aux.at[pl.ds(_TOTS + tid * 16, 16)])
        plsc.subcore_barrier()

        pltpu.sync_copy(sh_aux.at[pl.ds(_TOTS, 256)], tot_v)
        totals = plsc.load_gather(tot_v, [lio * 16])
        csum = plsc.cumsum(totals)
        excl = csum - totals
        hit = jnp.logical_and(csum >= rank, excl < rank)
        strip_id = jnp.sum(jnp.where(hit, lio, 0))
        below_strip = jnp.sum(jnp.where(hit, excl, 0))

        @pl.when(tid == strip_id)
        def _():
            cum = below_strip
            b_acc = jnp.int32(0)
            below_acc = jnp.int32(0)
            for ch in range(8):
                h = merged_v[pl.ds(ch * 16, 16)]
                cc = plsc.cumsum(h)
                cumv = cum + cc
                excl2 = cumv - h
                hit2 = jnp.logical_and(cumv >= rank, excl2 < rank)
                b_acc = b_acc + jnp.sum(
                    jnp.where(hit2, tid * 128 + ch * 16 + lio, 0))
                below_acc = below_acc + jnp.sum(jnp.where(hit2, excl2, 0))
                cum = cum + jnp.sum(h)
            rv = jnp.where(lio == 0, b_acc + zeros_i, zeros_i)
            rv = jnp.where(lio == 1, below_acc + zeros_i, rv)
            stage_v[...] = rv
            pltpu.sync_copy(stage_v, sh_aux.at[pl.ds(_RES, 16)])

        plsc.subcore_barrier()
        pltpu.sync_copy(sh_aux.at[pl.ds(_RES, 16)], res_v)
        rv = res_v[...]
        b = jnp.sum(jnp.where(lio == 0, rv, 0))
        below = jnp.sum(jnp.where(lio == 1, rv, 0))
        return b, below

    b1, below1 = hist_pass(0, None, None, jnp.int32(_RANK))
    p1 = b1 - 1024
    rank2 = jnp.int32(_RANK) - below1

    b2, below2 = hist_pass(1, p1, None, rank2)
    p2 = (p1 << 11) | b2
    rank3 = rank2 - below2

    b3, _ = hist_pass(2, None, p2, rank3)
    key_fin = (p2 << 10) | b3

    kf = key_fin + zeros_i
    tb = jnp.where(kf >= 0, kf, kf ^ jnp.int32(0x7FFFFFFF))
    tnll = plsc.bitcast(tb, jnp.float32)
    thr = jnp.minimum(tnll, jnp.float32(_NEG_LOG_THRESH))

    def loss_body(i, carry):
        s, cnt = carry
        for u in range(_UNROLL):
            v = data_v[pl.ds(i * (16 * _UNROLL) + u * 16, 16)]
            kept = v >= thr
            s = s + jnp.where(kept, v, 0.0)
            cnt = cnt + jnp.where(kept, 1, 0)
        return (s, cnt)

    s_v, c_v = lax.fori_loop(
        0, _NV // _UNROLL, loss_body,
        (jnp.zeros((16,), jnp.float32), zeros_i))

    stage_v[...] = lax.bitcast_convert_type(s_v, jnp.int32)
    pltpu.sync_copy(stage_v, sh_aux.at[pl.ds(_SUMS + tid * 16, 16)])
    stage_v[...] = c_v
    pltpu.sync_copy(stage_v, sh_aux.at[pl.ds(_CNTS + tid * 16, 16)])
    plsc.subcore_barrier()

    @pl.when(tid == 0)
    def _():
        pltpu.sync_copy(sh_aux.at[pl.ds(_SUMS, 256)], tot_v)
        sf = zeros_f
        for t in range(_NT):
            sf = sf + lax.bitcast_convert_type(
                tot_v[pl.ds(t * 16, 16)], jnp.float32)
        pltpu.sync_copy(sh_aux.at[pl.ds(_CNTS, 256)], tot_v)
        ci = zeros_i
        for t in range(_NT):
            ci = ci + tot_v[pl.ds(t * 16, 16)]
        num_v = jnp.sum(sf) + zeros_f
        cvec = jnp.sum(ci) + zeros_i
        den_v = jnp.maximum(cvec.astype(jnp.float32), 1.0)
        out_v[...] = num_v / den_v
        pltpu.sync_copy(out_v, out_hbm)


def kernel(pred, target):
    b, c, h, w = pred.shape
    hc = 16

    nll = pl.pallas_call(
        _nll_kernel,
        grid=(h // hc,),
        in_specs=[
            pl.BlockSpec((b, c, hc, w), lambda i: (0, 0, i, 0)),
            pl.BlockSpec((b, hc, w), lambda i: (0, i, 0)),
        ],
        out_specs=pl.BlockSpec((b, hc, w), lambda i: (0, i, 0)),
        out_shape=jax.ShapeDtypeStruct((b, h, w), jnp.float32),
    )(pred, target)

    mesh = plsc.VectorSubcoreMesh(
        core_axis_name="c", subcore_axis_name="s", num_cores=1)
    sc_select = pl.kernel(
        _sc_select_body,
        mesh=mesh,
        compiler_params=pltpu.CompilerParams(needs_layout_passes=False),
        out_type=jax.ShapeDtypeStruct((16,), jnp.float32),
        scratch_types=[
            pltpu.VMEM((_PER,), jnp.float32),
            pltpu.VMEM((_NBIN,), jnp.int32),
            pltpu.VMEM((16,), jnp.float32),
            pltpu.VMEM((_NT, 128), jnp.int32),
            pltpu.VMEM((128,), jnp.int32),
            pltpu.VMEM((16,), jnp.int32),
            pltpu.VMEM((256,), jnp.int32),
            pltpu.VMEM((16,), jnp.int32),
            pltpu.VMEM_SHARED((_NT, _NBIN), jnp.int32),
            pltpu.VMEM_SHARED((1024,), jnp.int32),
        ],
    )
    loss16 = sc_select(nll.reshape(-1))
    return loss16[0]


# speedup vs baseline: 1.1456x; 1.0364x over previous
import functools
import math
import struct

import jax
import jax.numpy as jnp
from jax import lax
from jax.experimental import pallas as pl
from jax.experimental.pallas import tpu as pltpu
from jax.experimental.pallas import tpu_sc as plsc

_THRESH = 0.6
_MIN_KEPT = 100000

_NEG_LOG_THRESH = -math.log(_THRESH)
_KEY06 = struct.unpack("<i", struct.pack("<f", _NEG_LOG_THRESH))[0]

_N = 4 * 512 * 512
_RANK = _N - _MIN_KEPT + 1
_NT = 16
_PER = _N // _NT
_NV = _PER // 16
_NBIN = 2048
_UNROLL = 8


def _nll_kernel(pred_ref, tgt_ref, nll_ref):
    x = pred_ref[...]
    m = jnp.max(x, axis=1, keepdims=True)
    sh = x - m
    s = jnp.sum(jnp.exp(sh), axis=1)
    t = tgt_ref[...]
    cls = jax.lax.broadcasted_iota(jnp.int32, x.shape, 1)
    sh_t = jnp.sum(jnp.where(cls == t[:, None], sh, 0.0), axis=1)
    nll_ref[...] = jnp.log(s) - sh_t


def _keys(v):
    b = lax.bitcast_convert_type(v, jnp.int32)
    return b ^ ((b >> 31) & jnp.int32(0x7FFFFFFF))


def _sc_select_body(nll_hbm, out_hbm, data_v, hist_v, out_v,
                    strip_v, merged_v, stage_v, tot_v, res_v,
                    sh_hist, sh_aux):
    tid = lax.axis_index("s")
    lio = lax.iota(jnp.int32, 16)
    zeros_i = jnp.zeros((16,), jnp.int32)
    ones_i = jnp.ones((16,), jnp.int32)
    zeros_f = jnp.zeros((16,), jnp.float32)
    _ATOT, _ACSUM, _ACCNT, _ASBEL, _TOTS, _RES = 0, 256, 512, 768, 1024, 1280

    pltpu.sync_copy(nll_hbm.at[pl.ds(tid * _PER, _PER)], data_v)

    def zero_hist(i, _):
        hist_v[pl.ds(i * 16, 16)] = zeros_i
        return 0

    acc = {}

    def hist_pass(rnd, p1, p2, rank):
        lax.fori_loop(0, _NBIN // 16, zero_hist, 0)

        if rnd == 0:
            def body0(i, carry):
                tot, csum, ccnt = carry
                for u in range(_UNROLL):
                    v = data_v[pl.ds(i * (16 * _UNROLL) + u * 16, 16)]
                    key = _keys(v)
                    bin_ = (key >> 21) + 1024
                    plsc.addupdate_scatter(hist_v, [bin_], ones_i)
                    lt = key < jnp.int32(_KEY06)
                    tot = tot + v
                    csum = csum + jnp.where(lt, v, 0.0)
                    ccnt = ccnt + jnp.where(lt, 1, 0)
                return (tot, csum, ccnt)

            acc["tot"], acc["csum"], acc["ccnt"] = lax.fori_loop(
                0, _NV // _UNROLL, body0, (zeros_f, zeros_f, zeros_i))
        else:
            def body(i, _):
                for u in range(_UNROLL):
                    key = _keys(
                        data_v[pl.ds(i * (16 * _UNROLL) + u * 16, 16)])
                    if rnd == 1:
                        bin_ = (key >> 10) & jnp.int32(0x7FF)
                        plsc.addupdate_scatter(hist_v, [bin_], ones_i,
                                               mask=(key >> 21) == p1)
                    else:
                        bin_ = key & jnp.int32(0x3FF)
                        plsc.addupdate_scatter(hist_v, [bin_], ones_i,
                                               mask=(key >> 10) == p2)
                return 0

            lax.fori_loop(0, _NV // _UNROLL, body, 0)

        pltpu.sync_copy(hist_v, sh_hist.at[tid])
        plsc.subcore_barrier()
        pltpu.sync_copy(sh_hist.at[:, pl.ds(tid * 128, 128)], strip_v)
        stot = jnp.int32(0)
        for ch in range(8):
            h = zeros_i
            for r in range(_NT):
                h = h + strip_v[r, pl.ds(ch * 16, 16)]
            merged_v[pl.ds(ch * 16, 16)] = h
            stot = stot + jnp.sum(h)
        stage_v[...] = stot + zeros_i
        pltpu.sync_copy(stage_v, sh_aux.at[pl.ds(_TOTS + tid * 16, 16)])
        plsc.subcore_barrier()

        pltpu.sync_copy(sh_aux.at[pl.ds(_TOTS, 256)], tot_v)
        totals = plsc.load_gather(tot_v, [lio * 16])
        csum = plsc.cumsum(totals)
        excl = csum - totals
        hit = jnp.logical_and(csum >= rank, excl < rank)
        strip_id = jnp.sum(jnp.where(hit, lio, 0))
        below_strip = jnp.sum(jnp.where(hit, excl, 0))

        @pl.when(tid == strip_id)
        def _():
            cum = below_strip
            b_acc = jnp.int32(0)
            below_acc = jnp.int32(0)
            for ch in range(8):
                h = merged_v[pl.ds(ch * 16, 16)]
                cc = plsc.cumsum(h)
                cumv = cum + cc
                excl2 = cumv - h
                hit2 = jnp.logical_and(cumv >= rank, excl2 < rank)
                b_acc = b_acc + jnp.sum(
                    jnp.where(hit2, tid * 128 + ch * 16 + lio, 0))
                below_acc = below_acc + jnp.sum(jnp.where(hit2, excl2, 0))
                cum = cum + jnp.sum(h)
            rv = jnp.where(lio == 0, b_acc + zeros_i, zeros_i)
            rv = jnp.where(lio == 1, below_acc + zeros_i, rv)
            stage_v[...] = rv
            pltpu.sync_copy(stage_v, sh_aux.at[pl.ds(_RES, 16)])

        plsc.subcore_barrier()
        pltpu.sync_copy(sh_aux.at[pl.ds(_RES, 16)], res_v)
        rv = res_v[...]
        b = jnp.sum(jnp.where(lio == 0, rv, 0))
        below = jnp.sum(jnp.where(lio == 1, rv, 0))
        return b, below

    b1, below1 = hist_pass(0, None, None, jnp.int32(_RANK))
    p1 = b1 - 1024
    rank2 = jnp.int32(_RANK) - below1

    b2, below2 = hist_pass(1, p1, None, rank2)
    p2 = (p1 << 11) | b2
    rank3 = rank2 - below2

    b3, below3 = hist_pass(2, None, p2, rank3)
    key_fin = (p2 << 10) | b3

    use_tnll = key_fin < jnp.int32(_KEY06)

    def below_pass():
        def b(i, s):
            for u in range(_UNROLL):
                v = data_v[pl.ds(i * (16 * _UNROLL) + u * 16, 16)]
                key = _keys(v)
                s = s + jnp.where(key < key_fin, v, 0.0)
            return s
        return lax.fori_loop(0, _NV // _UNROLL, b, zeros_f)

    sbel_v = lax.cond(use_tnll, below_pass, lambda: zeros_f)

    for off, vec in ((_ATOT, lax.bitcast_convert_type(acc["tot"], jnp.int32)),
                     (_ACSUM, lax.bitcast_convert_type(acc["csum"],
                                                       jnp.int32)),
                     (_ACCNT, acc["ccnt"]),
                     (_ASBEL, lax.bitcast_convert_type(sbel_v, jnp.int32))):
        stage_v[...] = vec
        pltpu.sync_copy(stage_v, sh_aux.at[pl.ds(off + tid * 16, 16)])
    plsc.subcore_barrier()

    @pl.when(tid == 0)
    def _():
        def red_f(off):
            pltpu.sync_copy(sh_aux.at[pl.ds(off, 256)], tot_v)
            sf = zeros_f
            for t in range(_NT):
                sf = sf + lax.bitcast_convert_type(
                    tot_v[pl.ds(t * 16, 16)], jnp.float32)
            return jnp.sum(sf)

        tot_s = red_f(_ATOT)
        csum_s = red_f(_ACSUM)
        sbel_s = red_f(_ASBEL)
        pltpu.sync_copy(sh_aux.at[pl.ds(_ACCNT, 256)], tot_v)
        ci = zeros_i
        for t in range(_NT):
            ci = ci + tot_v[pl.ds(t * 16, 16)]
        ccnt_s = jnp.sum(ci)

        strictly_below = (jnp.int32(_RANK) - rank3) + below3
        cnt = jnp.where(use_tnll, jnp.int32(_N) - strictly_below,
                        jnp.int32(_N) - ccnt_s)
        num = jnp.where(use_tnll, tot_s - sbel_s, tot_s - csum_s)
        num_v = num + zeros_f
        cvec = cnt + zeros_i
        den_v = jnp.maximum(cvec.astype(jnp.float32), 1.0)
        out_v[...] = num_v / den_v
        pltpu.sync_copy(out_v, out_hbm)


def kernel(pred, target):
    b, c, h, w = pred.shape
    hc = 16

    nll = pl.pallas_call(
        _nll_kernel,
        grid=(h // hc,),
        in_specs=[
            pl.BlockSpec((b, c, hc, w), lambda i: (0, 0, i, 0)),
            pl.BlockSpec((b, hc, w), lambda i: (0, i, 0)),
        ],
        out_specs=pl.BlockSpec((b, hc, w), lambda i: (0, i, 0)),
        out_shape=jax.ShapeDtypeStruct((b, h, w), jnp.float32),
    )(pred, target)

    mesh = plsc.VectorSubcoreMesh(
        core_axis_name="c", subcore_axis_name="s", num_cores=1)
    sc_select = pl.kernel(
        _sc_select_body,
        mesh=mesh,
        compiler_params=pltpu.CompilerParams(needs_layout_passes=False),
        out_type=jax.ShapeDtypeStruct((16,), jnp.float32),
        scratch_types=[
            pltpu.VMEM((_PER,), jnp.float32),
            pltpu.VMEM((_NBIN,), jnp.int32),
            pltpu.VMEM((16,), jnp.float32),
            pltpu.VMEM((_NT, 128), jnp.int32),
            pltpu.VMEM((128,), jnp.int32),
            pltpu.VMEM((16,), jnp.int32),
            pltpu.VMEM((256,), jnp.int32),
            pltpu.VMEM((16,), jnp.int32),
            pltpu.VMEM_SHARED((_NT, _NBIN), jnp.int32),
            pltpu.VMEM_SHARED((2048,), jnp.int32),
        ],
    )
    loss16 = sc_select(nll.reshape(-1))
    return loss16[0]


# speedup vs baseline: 2.2599x; 1.9726x over previous
import functools
import math
import struct

import jax
import jax.numpy as jnp
from jax import lax
from jax.experimental import pallas as pl
from jax.experimental.pallas import tpu as pltpu
from jax.experimental.pallas import tpu_sc as plsc

_THRESH = 0.6
_MIN_KEPT = 100000

_NEG_LOG_THRESH = -math.log(_THRESH)
_KEY06 = struct.unpack("<i", struct.pack("<f", _NEG_LOG_THRESH))[0]

_N = 4 * 512 * 512
_RANK = _N - _MIN_KEPT + 1
_NT = 16
_PER = _N // _NT
_NV = _PER // 16
_NBIN = 2048
_UNROLL = 8


def _nll_kernel(pred_ref, tgt_ref, nll_ref):
    x = pred_ref[...]
    m = jnp.max(x, axis=1, keepdims=True)
    sh = x - m
    s = jnp.sum(jnp.exp(sh), axis=1)
    t = tgt_ref[...]
    cls = jax.lax.broadcasted_iota(jnp.int32, x.shape, 1)
    sh_t = jnp.sum(jnp.where(cls == t[:, None], sh, 0.0), axis=1)
    nll_ref[...] = jnp.log(s) - sh_t


def _keys(v):
    b = lax.bitcast_convert_type(v, jnp.int32)
    return b ^ ((b >> 31) & jnp.int32(0x7FFFFFFF))


def _sc_select_body(nll_hbm, out_hbm, data_v, hist_v, out_v,
                    strip_v, merged_v, stage_v, tot_v, res_v,
                    sh_hist, sh_aux):
    tid = lax.axis_index("s")
    lio = lax.iota(jnp.int32, 16)
    zeros_i = jnp.zeros((16,), jnp.int32)
    ones_i = jnp.ones((16,), jnp.int32)
    zeros_f = jnp.zeros((16,), jnp.float32)
    _ATOT, _ACSUM, _ACCNT, _ASBEL, _TOTS, _RES = 0, 256, 512, 768, 1024, 1280

    pltpu.sync_copy(nll_hbm.at[pl.ds(tid * _PER, _PER)], data_v)

    def zero_hist(i, _):
        hist_v[pl.ds(i * 16, 16)] = zeros_i
        return 0

    acc = {}

    def hist_pass(rnd, p1, p2, rank):
        lax.fori_loop(0, _NBIN // 16, zero_hist, 0)

        if rnd == 0:
            @plsc.parallel_loop(0, _NV, 1, unroll=_UNROLL,
                                carry=(zeros_f, zeros_f, zeros_i))
            def body0(i, carry):
                tot, csum, ccnt = carry
                v = data_v[pl.ds(i * 16, 16)]
                key = _keys(v)
                bin_ = (key >> 21) + 1024
                plsc.addupdate_scatter(hist_v, [bin_], ones_i)
                lt = key < jnp.int32(_KEY06)
                return (tot + v, csum + jnp.where(lt, v, 0.0),
                        ccnt + jnp.where(lt, 1, 0))

            acc["tot"], acc["csum"], acc["ccnt"] = body0
        else:
            @plsc.parallel_loop(0, _NV, 1, unroll=_UNROLL)
            def _(i):
                key = _keys(data_v[pl.ds(i * 16, 16)])
                if rnd == 1:
                    bin_ = (key >> 10) & jnp.int32(0x7FF)
                    plsc.addupdate_scatter(hist_v, [bin_], ones_i,
                                           mask=(key >> 21) == p1)
                else:
                    bin_ = key & jnp.int32(0x3FF)
                    plsc.addupdate_scatter(hist_v, [bin_], ones_i,
                                           mask=(key >> 10) == p2)

        pltpu.sync_copy(hist_v, sh_hist.at[tid])
        plsc.subcore_barrier()
        pltpu.sync_copy(sh_hist.at[:, pl.ds(tid * 128, 128)], strip_v)
        stot = jnp.int32(0)
        for ch in range(8):
            h = zeros_i
            for r in range(_NT):
                h = h + strip_v[r, pl.ds(ch * 16, 16)]
            merged_v[pl.ds(ch * 16, 16)] = h
            stot = stot + jnp.sum(h)
        stage_v[...] = stot + zeros_i
        pltpu.sync_copy(stage_v, sh_aux.at[pl.ds(_TOTS + tid * 16, 16)])
        plsc.subcore_barrier()

        pltpu.sync_copy(sh_aux.at[pl.ds(_TOTS, 256)], tot_v)
        totals = plsc.load_gather(tot_v, [lio * 16])
        csum = plsc.cumsum(totals)
        excl = csum - totals
        hit = jnp.logical_and(csum >= rank, excl < rank)
        strip_id = jnp.sum(jnp.where(hit, lio, 0))
        below_strip = jnp.sum(jnp.where(hit, excl, 0))

        @pl.when(tid == strip_id)
        def _():
            cum = below_strip
            b_acc = jnp.int32(0)
            below_acc = jnp.int32(0)
            for ch in range(8):
                h = merged_v[pl.ds(ch * 16, 16)]
                cc = plsc.cumsum(h)
                cumv = cum + cc
                excl2 = cumv - h
                hit2 = jnp.logical_and(cumv >= rank, excl2 < rank)
                b_acc = b_acc + jnp.sum(
                    jnp.where(hit2, tid * 128 + ch * 16 + lio, 0))
                below_acc = below_acc + jnp.sum(jnp.where(hit2, excl2, 0))
                cum = cum + jnp.sum(h)
            rv = jnp.where(lio == 0, b_acc + zeros_i, zeros_i)
            rv = jnp.where(lio == 1, below_acc + zeros_i, rv)
            stage_v[...] = rv
            pltpu.sync_copy(stage_v, sh_aux.at[pl.ds(_RES, 16)])

        plsc.subcore_barrier()
        pltpu.sync_copy(sh_aux.at[pl.ds(_RES, 16)], res_v)
        rv = res_v[...]
        b = jnp.sum(jnp.where(lio == 0, rv, 0))
        below = jnp.sum(jnp.where(lio == 1, rv, 0))
        return b, below

    b1, below1 = hist_pass(0, None, None, jnp.int32(_RANK))
    p1 = b1 - 1024
    rank2 = jnp.int32(_RANK) - below1

    b2, below2 = hist_pass(1, p1, None, rank2)
    p2 = (p1 << 11) | b2
    rank3 = rank2 - below2

    b3, below3 = hist_pass(2, None, p2, rank3)
    key_fin = (p2 << 10) | b3

    use_tnll = key_fin < jnp.int32(_KEY06)

    def below_pass():
        @plsc.parallel_loop(0, _NV, 1, unroll=_UNROLL, carry=zeros_f)
        def b(i, s):
            v = data_v[pl.ds(i * 16, 16)]
            key = _keys(v)
            return s + jnp.where(key < key_fin, v, 0.0)
        return b

    sbel_v = lax.cond(use_tnll, below_pass, lambda: zeros_f)

    for off, vec in ((_ATOT, lax.bitcast_convert_type(acc["tot"], jnp.int32)),
                     (_ACSUM, lax.bitcast_convert_type(acc["csum"],
                                                       jnp.int32)),
                     (_ACCNT, acc["ccnt"]),
                     (_ASBEL, lax.bitcast_convert_type(sbel_v, jnp.int32))):
        stage_v[...] = vec
        pltpu.sync_copy(stage_v, sh_aux.at[pl.ds(off + tid * 16, 16)])
    plsc.subcore_barrier()

    @pl.when(tid == 0)
    def _():
        def red_f(off):
            pltpu.sync_copy(sh_aux.at[pl.ds(off, 256)], tot_v)
            sf = zeros_f
            for t in range(_NT):
                sf = sf + lax.bitcast_convert_type(
                    tot_v[pl.ds(t * 16, 16)], jnp.float32)
            return jnp.sum(sf)

        tot_s = red_f(_ATOT)
        csum_s = red_f(_ACSUM)
        sbel_s = red_f(_ASBEL)
        pltpu.sync_copy(sh_aux.at[pl.ds(_ACCNT, 256)], tot_v)
        ci = zeros_i
        for t in range(_NT):
            ci = ci + tot_v[pl.ds(t * 16, 16)]
        ccnt_s = jnp.sum(ci)

        strictly_below = (jnp.int32(_RANK) - rank3) + below3
        cnt = jnp.where(use_tnll, jnp.int32(_N) - strictly_below,
                        jnp.int32(_N) - ccnt_s)
        num = jnp.where(use_tnll, tot_s - sbel_s, tot_s - csum_s)
        num_v = num + zeros_f
        cvec = cnt + zeros_i
        den_v = jnp.maximum(cvec.astype(jnp.float32), 1.0)
        out_v[...] = num_v / den_v
        pltpu.sync_copy(out_v, out_hbm)


def kernel(pred, target):
    b, c, h, w = pred.shape
    hc = 16

    nll = pl.pallas_call(
        _nll_kernel,
        grid=(h // hc,),
        in_specs=[
            pl.BlockSpec((b, c, hc, w), lambda i: (0, 0, i, 0)),
            pl.BlockSpec((b, hc, w), lambda i: (0, i, 0)),
        ],
        out_specs=pl.BlockSpec((b, hc, w), lambda i: (0, i, 0)),
        out_shape=jax.ShapeDtypeStruct((b, h, w), jnp.float32),
    )(pred, target)

    mesh = plsc.VectorSubcoreMesh(
        core_axis_name="c", subcore_axis_name="s", num_cores=1)
    sc_select = pl.kernel(
        _sc_select_body,
        mesh=mesh,
        compiler_params=pltpu.CompilerParams(needs_layout_passes=False),
        out_type=jax.ShapeDtypeStruct((16,), jnp.float32),
        scratch_types=[
            pltpu.VMEM((_PER,), jnp.float32),
            pltpu.VMEM((_NBIN,), jnp.int32),
            pltpu.VMEM((16,), jnp.float32),
            pltpu.VMEM((_NT, 128), jnp.int32),
            pltpu.VMEM((128,), jnp.int32),
            pltpu.VMEM((16,), jnp.int32),
            pltpu.VMEM((256,), jnp.int32),
            pltpu.VMEM((16,), jnp.int32),
            pltpu.VMEM_SHARED((_NT, _NBIN), jnp.int32),
            pltpu.VMEM_SHARED((2048,), jnp.int32),
        ],
    )
    loss16 = sc_select(nll.reshape(-1))
    return loss16[0]


# speedup vs baseline: 2.4654x; 1.0909x over previous
import functools
import math
import struct

import jax
import jax.numpy as jnp
from jax import lax
from jax.experimental import pallas as pl
from jax.experimental.pallas import tpu as pltpu
from jax.experimental.pallas import tpu_sc as plsc

_THRESH = 0.6
_MIN_KEPT = 100000

_NEG_LOG_THRESH = -math.log(_THRESH)
_KEY06 = struct.unpack("<i", struct.pack("<f", _NEG_LOG_THRESH))[0]

_N = 4 * 512 * 512
_RANK = _N - _MIN_KEPT + 1
_NT = 16
_PER = _N // _NT
_NV = _PER // 16
_NBIN = 2048
_UNROLL = 8


def _nll_kernel(pred_ref, tgt_ref, nll_ref):
    x = pred_ref[...]
    m = jnp.max(x, axis=1, keepdims=True)
    sh = x - m
    s = jnp.sum(jnp.exp(sh), axis=1)
    t = tgt_ref[...]
    cls = jax.lax.broadcasted_iota(jnp.int32, x.shape, 1)
    sh_t = jnp.sum(jnp.where(cls == t[:, None], sh, 0.0), axis=1)
    nll_ref[...] = jnp.log(s) - sh_t


def _keys(v):
    b = lax.bitcast_convert_type(v, jnp.int32)
    return b ^ ((b >> 31) & jnp.int32(0x7FFFFFFF))


def _sc_select_body(nll_hbm, out_hbm, data_v, hist_v, out_v,
                    strip_v, merged_v, stage_v, tot_v, res_v,
                    sh_hist, sh_aux):
    tid = lax.axis_index("s")
    lio = lax.iota(jnp.int32, 16)
    zeros_i = jnp.zeros((16,), jnp.int32)
    ones_i = jnp.ones((16,), jnp.int32)
    zeros_f = jnp.zeros((16,), jnp.float32)
    _ATOT, _ACSUM, _ACCNT, _ASBEL, _TOTS, _RES = 0, 256, 512, 768, 1024, 1280

    pltpu.sync_copy(nll_hbm.at[pl.ds(tid * _PER, _PER)], data_v)

    def zero_hist(i, _):
        hist_v[pl.ds(i * 16, 16)] = zeros_i
        return 0

    acc = {}

    def hist_pass(rnd, p1, p2, rank):
        lax.fori_loop(0, _NBIN // 16, zero_hist, 0)

        if rnd == 0:
            @plsc.parallel_loop(0, _NV, 1, unroll=_UNROLL,
                                carry=(zeros_f, zeros_f, zeros_i))
            def body0(i, carry):
                tot, csum, ccnt = carry
                v = data_v[pl.ds(i * 16, 16)]
                key = _keys(v)
                bin_ = (key >> 21) + 1024
                plsc.addupdate_scatter(hist_v, [bin_], ones_i)
                lt = key < jnp.int32(_KEY06)
                return (tot + v, csum + jnp.where(lt, v, 0.0),
                        ccnt + jnp.where(lt, 1, 0))

            acc["tot"], acc["csum"], acc["ccnt"] = body0
        else:
            @plsc.parallel_loop(0, _NV, 1, unroll=_UNROLL)
            def _(i):
                key = _keys(data_v[pl.ds(i * 16, 16)])
                if rnd == 1:
                    bin_ = (key >> 10) & jnp.int32(0x7FF)
                    plsc.addupdate_scatter(hist_v, [bin_], ones_i,
                                           mask=(key >> 21) == p1)
                else:
                    bin_ = key & jnp.int32(0x3FF)
                    plsc.addupdate_scatter(hist_v, [bin_], ones_i,
                                           mask=(key >> 10) == p2)

        pltpu.sync_copy(hist_v, sh_hist.at[tid])
        plsc.subcore_barrier()
        pltpu.sync_copy(sh_hist.at[:, pl.ds(tid * 128, 128)], strip_v)
        stot = jnp.int32(0)
        for ch in range(8):
            h = zeros_i
            for r in range(_NT):
                h = h + strip_v[r, pl.ds(ch * 16, 16)]
            merged_v[pl.ds(ch * 16, 16)] = h
            stot = stot + jnp.sum(h)
        stage_v[...] = stot + zeros_i
        pltpu.sync_copy(stage_v, sh_aux.at[pl.ds(_TOTS + tid * 16, 16)])
        plsc.subcore_barrier()

        pltpu.sync_copy(sh_aux.at[pl.ds(_TOTS, 256)], tot_v)
        totals = plsc.load_gather(tot_v, [lio * 16])
        csum = plsc.cumsum(totals)
        excl = csum - totals
        hit = jnp.logical_and(csum >= rank, excl < rank)
        strip_id = jnp.sum(jnp.where(hit, lio, 0))
        below_strip = jnp.sum(jnp.where(hit, excl, 0))

        @pl.when(tid == strip_id)
        def _():
            cum = below_strip
            b_acc = jnp.int32(0)
            below_acc = jnp.int32(0)
            for ch in range(8):
                h = merged_v[pl.ds(ch * 16, 16)]
                cc = plsc.cumsum(h)
                cumv = cum + cc
                excl2 = cumv - h
                hit2 = jnp.logical_and(cumv >= rank, excl2 < rank)
                b_acc = b_acc + jnp.sum(
                    jnp.where(hit2, tid * 128 + ch * 16 + lio, 0))
                below_acc = below_acc + jnp.sum(jnp.where(hit2, excl2, 0))
                cum = cum + jnp.sum(h)
            rv = jnp.where(lio == 0, b_acc + zeros_i, zeros_i)
            rv = jnp.where(lio == 1, below_acc + zeros_i, rv)
            stage_v[...] = rv
            pltpu.sync_copy(stage_v, sh_aux.at[pl.ds(_RES, 16)])

        plsc.subcore_barrier()
        pltpu.sync_copy(sh_aux.at[pl.ds(_RES, 16)], res_v)
        rv = res_v[...]
        b = jnp.sum(jnp.where(lio == 0, rv, 0))
        below = jnp.sum(jnp.where(lio == 1, rv, 0))
        return b, below

    b1, below1 = hist_pass(0, None, None, jnp.int32(_RANK))
    p1 = b1 - 1024
    rank2 = jnp.int32(_RANK) - below1

    b2, below2 = hist_pass(1, p1, None, rank2)
    p2 = (p1 << 11) | b2
    rank3 = rank2 - below2

    b3, below3 = hist_pass(2, None, p2, rank3)
    key_fin = (p2 << 10) | b3

    use_tnll = key_fin < jnp.int32(_KEY06)

    def below_pass():
        @plsc.parallel_loop(0, _NV, 1, unroll=_UNROLL, carry=zeros_f)
        def b(i, s):
            v = data_v[pl.ds(i * 16, 16)]
            key = _keys(v)
            return s + jnp.where(key < key_fin, v, 0.0)
        return b

    sbel_v = lax.cond(use_tnll, below_pass, lambda: zeros_f)

    for off, vec in ((_ATOT, lax.bitcast_convert_type(acc["tot"], jnp.int32)),
                     (_ACSUM, lax.bitcast_convert_type(acc["csum"],
                                                       jnp.int32)),
                     (_ACCNT, acc["ccnt"]),
                     (_ASBEL, lax.bitcast_convert_type(sbel_v, jnp.int32))):
        stage_v[...] = vec
        pltpu.sync_copy(stage_v, sh_aux.at[pl.ds(off + tid * 16, 16)])
    plsc.subcore_barrier()

    @pl.when(tid == 0)
    def _():
        def red_f(off):
            pltpu.sync_copy(sh_aux.at[pl.ds(off, 256)], tot_v)
            sf = zeros_f
            for t in range(_NT):
                sf = sf + lax.bitcast_convert_type(
                    tot_v[pl.ds(t * 16, 16)], jnp.float32)
            return jnp.sum(sf)

        tot_s = red_f(_ATOT)
        csum_s = red_f(_ACSUM)
        sbel_s = red_f(_ASBEL)
        pltpu.sync_copy(sh_aux.at[pl.ds(_ACCNT, 256)], tot_v)
        ci = zeros_i
        for t in range(_NT):
            ci = ci + tot_v[pl.ds(t * 16, 16)]
        ccnt_s = jnp.sum(ci)

        strictly_below = (jnp.int32(_RANK) - rank3) + below3
        cnt = jnp.where(use_tnll, jnp.int32(_N) - strictly_below,
                        jnp.int32(_N) - ccnt_s)
        num = jnp.where(use_tnll, tot_s - sbel_s, tot_s - csum_s)
        num_v = num + zeros_f
        cvec = cnt + zeros_i
        den_v = jnp.maximum(cvec.astype(jnp.float32), 1.0)
        out_v[...] = num_v / den_v
        pltpu.sync_copy(out_v, out_hbm)


def kernel(pred, target):
    b, c, h, w = pred.shape
    hc = 32

    nll = pl.pallas_call(
        _nll_kernel,
        grid=(h // hc,),
        in_specs=[
            pl.BlockSpec((b, c, hc, w), lambda i: (0, 0, i, 0)),
            pl.BlockSpec((b, hc, w), lambda i: (0, i, 0)),
        ],
        out_specs=pl.BlockSpec((b, hc, w), lambda i: (0, i, 0)),
        out_shape=jax.ShapeDtypeStruct((b, h, w), jnp.float32),
    )(pred, target)

    mesh = plsc.VectorSubcoreMesh(
        core_axis_name="c", subcore_axis_name="s", num_cores=1)
    sc_select = pl.kernel(
        _sc_select_body,
        mesh=mesh,
        compiler_params=pltpu.CompilerParams(needs_layout_passes=False),
        out_type=jax.ShapeDtypeStruct((16,), jnp.float32),
        scratch_types=[
            pltpu.VMEM((_PER,), jnp.float32),
            pltpu.VMEM((_NBIN,), jnp.int32),
            pltpu.VMEM((16,), jnp.float32),
            pltpu.VMEM((_NT, 128), jnp.int32),
            pltpu.VMEM((128,), jnp.int32),
            pltpu.VMEM((16,), jnp.int32),
            pltpu.VMEM((256,), jnp.int32),
            pltpu.VMEM((16,), jnp.int32),
            pltpu.VMEM_SHARED((_NT, _NBIN), jnp.int32),
            pltpu.VMEM_SHARED((2048,), jnp.int32),
        ],
    )
    loss16 = sc_select(nll.reshape(-1))
    return loss16[0]


# speedup vs baseline: 2.5418x; 1.0310x over previous
import functools
import math
import struct

import jax
import jax.numpy as jnp
from jax import lax
from jax.experimental import pallas as pl
from jax.experimental.pallas import tpu as pltpu
from jax.experimental.pallas import tpu_sc as plsc

_THRESH = 0.6
_MIN_KEPT = 100000

_NEG_LOG_THRESH = -math.log(_THRESH)
_KEY06 = struct.unpack("<i", struct.pack("<f", _NEG_LOG_THRESH))[0]

_N = 4 * 512 * 512
_RANK = _N - _MIN_KEPT + 1
_NT = 16
_PER = _N // _NT
_NV = _PER // 16
_NBIN = 2048
_UNROLL = 8


def _nll_kernel(pred_ref, tgt_ref, nll_ref):
    x = pred_ref[...]
    m = jnp.max(x, axis=1, keepdims=True)
    sh = x - m
    s = jnp.sum(jnp.exp(sh), axis=1)
    t = tgt_ref[...]
    cls = jax.lax.broadcasted_iota(jnp.int32, x.shape, 1)
    sh_t = jnp.sum(jnp.where(cls == t[:, None], sh, 0.0), axis=1)
    nll_ref[...] = jnp.log(s) - sh_t


def _keys(v):
    b = lax.bitcast_convert_type(v, jnp.int32)
    return b ^ ((b >> 31) & jnp.int32(0x7FFFFFFF))


def _sc_select_body(nll_hbm, out_hbm, data_v, hist_v, out_v,
                    strip_v, merged_v, stage_v, tot_v, res_v,
                    sh_hist, sh_aux):
    tid = lax.axis_index("s")
    lio = lax.iota(jnp.int32, 16)
    zeros_i = jnp.zeros((16,), jnp.int32)
    ones_i = jnp.ones((16,), jnp.int32)
    zeros_f = jnp.zeros((16,), jnp.float32)
    _ATOT, _ACSUM, _ACCNT, _ASBEL, _TOTS, _RES = 0, 256, 512, 768, 1024, 1280

    pltpu.sync_copy(nll_hbm.at[pl.ds(tid * _PER, _PER)], data_v)

    def zero_hist(i, _):
        hist_v[pl.ds(i * 16, 16)] = zeros_i
        return 0

    acc = {}

    def hist_pass(rnd, p1, p2, rank):
        lax.fori_loop(0, _NBIN // 16, zero_hist, 0)

        if rnd == 0:
            @plsc.parallel_loop(0, _NV, 1, unroll=_UNROLL,
                                carry=(zeros_f, zeros_f, zeros_i))
            def body0(i, carry):
                tot, csum, ccnt = carry
                v = data_v[pl.ds(i * 16, 16)]
                key = _keys(v)
                bin_ = (key >> 21) + 1024
                plsc.addupdate_scatter(hist_v, [bin_], ones_i)
                lt = key < jnp.int32(_KEY06)
                return (tot + v, csum + jnp.where(lt, v, 0.0),
                        ccnt + jnp.where(lt, 1, 0))

            acc["tot"], acc["csum"], acc["ccnt"] = body0
        else:
            @plsc.parallel_loop(0, _NV, 1, unroll=_UNROLL)
            def _(i):
                key = _keys(data_v[pl.ds(i * 16, 16)])
                if rnd == 1:
                    bin_ = (key >> 10) & jnp.int32(0x7FF)
                    plsc.addupdate_scatter(hist_v, [bin_], ones_i,
                                           mask=(key >> 21) == p1)
                else:
                    bin_ = key & jnp.int32(0x3FF)
                    plsc.addupdate_scatter(hist_v, [bin_], ones_i,
                                           mask=(key >> 10) == p2)

        pltpu.sync_copy(hist_v, sh_hist.at[tid])
        plsc.subcore_barrier()
        pltpu.sync_copy(sh_hist.at[:, pl.ds(tid * 128, 128)], strip_v)
        stot = jnp.int32(0)
        for ch in range(8):
            h = zeros_i
            for r in range(_NT):
                h = h + strip_v[r, pl.ds(ch * 16, 16)]
            merged_v[pl.ds(ch * 16, 16)] = h
            stot = stot + jnp.sum(h)
        stage_v[...] = stot + zeros_i
        pltpu.sync_copy(stage_v, sh_aux.at[pl.ds(_TOTS + tid * 16, 16)])
        plsc.subcore_barrier()

        pltpu.sync_copy(sh_aux.at[pl.ds(_TOTS, 256)], tot_v)
        totals = plsc.load_gather(tot_v, [lio * 16])
        csum = plsc.cumsum(totals)
        excl = csum - totals
        hit = jnp.logical_and(csum >= rank, excl < rank)
        strip_id = jnp.sum(jnp.where(hit, lio, 0))
        below_strip = jnp.sum(jnp.where(hit, excl, 0))

        @pl.when(tid == strip_id)
        def _():
            cum = below_strip
            b_acc = jnp.int32(0)
            below_acc = jnp.int32(0)
            for ch in range(8):
                h = merged_v[pl.ds(ch * 16, 16)]
                cc = plsc.cumsum(h)
                cumv = cum + cc
                excl2 = cumv - h
                hit2 = jnp.logical_and(cumv >= rank, excl2 < rank)
                b_acc = b_acc + jnp.sum(
                    jnp.where(hit2, tid * 128 + ch * 16 + lio, 0))
                below_acc = below_acc + jnp.sum(jnp.where(hit2, excl2, 0))
                cum = cum + jnp.sum(h)
            rv = jnp.where(lio == 0, b_acc + zeros_i, zeros_i)
            rv = jnp.where(lio == 1, below_acc + zeros_i, rv)
            stage_v[...] = rv
            pltpu.sync_copy(stage_v, sh_aux.at[pl.ds(_RES, 16)])

        plsc.subcore_barrier()
        pltpu.sync_copy(sh_aux.at[pl.ds(_RES, 16)], res_v)
        rv = res_v[...]
        b = jnp.sum(jnp.where(lio == 0, rv, 0))
        below = jnp.sum(jnp.where(lio == 1, rv, 0))
        return b, below

    b1, below1 = hist_pass(0, None, None, jnp.int32(_RANK))
    p1 = b1 - 1024
    rank2 = jnp.int32(_RANK) - below1

    b2, below2 = hist_pass(1, p1, None, rank2)
    p2 = (p1 << 11) | b2
    rank3 = rank2 - below2

    b3, below3 = hist_pass(2, None, p2, rank3)
    key_fin = (p2 << 10) | b3

    use_tnll = key_fin < jnp.int32(_KEY06)

    def below_pass():
        @plsc.parallel_loop(0, _NV, 1, unroll=_UNROLL, carry=zeros_f)
        def b(i, s):
            v = data_v[pl.ds(i * 16, 16)]
            key = _keys(v)
            return s + jnp.where(key < key_fin, v, 0.0)
        return b

    sbel_v = lax.cond(use_tnll, below_pass, lambda: zeros_f)

    for off, vec in ((_ATOT, lax.bitcast_convert_type(acc["tot"], jnp.int32)),
                     (_ACSUM, lax.bitcast_convert_type(acc["csum"],
                                                       jnp.int32)),
                     (_ACCNT, acc["ccnt"]),
                     (_ASBEL, lax.bitcast_convert_type(sbel_v, jnp.int32))):
        stage_v[...] = vec
        pltpu.sync_copy(stage_v, sh_aux.at[pl.ds(off + tid * 16, 16)])
    plsc.subcore_barrier()

    @pl.when(tid == 0)
    def _():
        def red_f(off):
            pltpu.sync_copy(sh_aux.at[pl.ds(off, 256)], tot_v)
            sf = zeros_f
            for t in range(_NT):
                sf = sf + lax.bitcast_convert_type(
                    tot_v[pl.ds(t * 16, 16)], jnp.float32)
            return jnp.sum(sf)

        tot_s = red_f(_ATOT)
        csum_s = red_f(_ACSUM)
        sbel_s = red_f(_ASBEL)
        pltpu.sync_copy(sh_aux.at[pl.ds(_ACCNT, 256)], tot_v)
        ci = zeros_i
        for t in range(_NT):
            ci = ci + tot_v[pl.ds(t * 16, 16)]
        ccnt_s = jnp.sum(ci)

        strictly_below = (jnp.int32(_RANK) - rank3) + below3
        cnt = jnp.where(use_tnll, jnp.int32(_N) - strictly_below,
                        jnp.int32(_N) - ccnt_s)
        num = jnp.where(use_tnll, tot_s - sbel_s, tot_s - csum_s)
        num_v = num + zeros_f
        cvec = cnt + zeros_i
        den_v = jnp.maximum(cvec.astype(jnp.float32), 1.0)
        out_v[...] = num_v / den_v
        pltpu.sync_copy(out_v, out_hbm)


def kernel(pred, target):
    b, c, h, w = pred.shape
    hc = 64

    nll = pl.pallas_call(
        _nll_kernel,
        grid=(h // hc,),
        in_specs=[
            pl.BlockSpec((b, c, hc, w), lambda i: (0, 0, i, 0)),
            pl.BlockSpec((b, hc, w), lambda i: (0, i, 0)),
        ],
        out_specs=pl.BlockSpec((b, hc, w), lambda i: (0, i, 0)),
        out_shape=jax.ShapeDtypeStruct((b, h, w), jnp.float32),
    )(pred, target)

    mesh = plsc.VectorSubcoreMesh(
        core_axis_name="c", subcore_axis_name="s", num_cores=1)
    sc_select = pl.kernel(
        _sc_select_body,
        mesh=mesh,
        compiler_params=pltpu.CompilerParams(needs_layout_passes=False),
        out_type=jax.ShapeDtypeStruct((16,), jnp.float32),
        scratch_types=[
            pltpu.VMEM((_PER,), jnp.float32),
            pltpu.VMEM((_NBIN,), jnp.int32),
            pltpu.VMEM((16,), jnp.float32),
            pltpu.VMEM((_NT, 128), jnp.int32),
            pltpu.VMEM((128,), jnp.int32),
            pltpu.VMEM((16,), jnp.int32),
            pltpu.VMEM((256,), jnp.int32),
            pltpu.VMEM((16,), jnp.int32),
            pltpu.VMEM_SHARED((_NT, _NBIN), jnp.int32),
            pltpu.VMEM_SHARED((2048,), jnp.int32),
        ],
    )
    loss16 = sc_select(nll.reshape(-1))
    return loss16[0]
